# baseline (speedup 1.0000x reference)
import functools

import jax
import jax.numpy as jnp
from jax import lax
from jax.experimental import pallas as pl
from jax.experimental.pallas import tpu as pltpu
from jax.experimental.pallas import tpu_sc as plsc

BATCH = 4096
SEQ = 200
DIM = 64
NBINS_X = 1000
NY = 201
NLANE = 8

NUM_CORES = 2
NUM_SUBCORES = 16
NUM_WORKERS = NUM_CORES * NUM_SUBCORES
ROWS_PER_W = BATCH // NUM_WORKERS
LANES = 16
DBLK = DIM // LANES
NGROUP = (SEQ + LANES - 1) // LANES
SEQ_PAD = NGROUP * LANES
DPAD = 128
PROWS = SEQ // 2
PBUF = SEQ_PAD // 2
TOKBUF = 2 * SEQ + LANES


def _body(
    xtok_hbm,
    ytok_hbm,
    lidx_hbm,
    xtab_hbm,
    ytab_hbm,
    pos_hbm,
    lane_hbm,
    out_hbm,
    xtab_v,
    ytab_v,
    pos_v,
    lane_v,
    lidx_v,
    tokx_v,
    toky_v,
    outq0_v,
    outq1_v,
    sem_q0,
    sem_q1,
    sem_t,
):
    outq = (outq0_v, outq1_v)
    sem_q = (sem_q0, sem_q1)
    wid = lax.axis_index("s") * NUM_CORES + lax.axis_index("c")
    base_b = wid * ROWS_PER_W

    zero16 = jnp.zeros((LANES,), jnp.int32)
    for t in range(2):
        tokx_v[pl.ds(t * TOKBUF + 2 * SEQ, LANES)] = zero16
        toky_v[pl.ds(t * TOKBUF + 2 * SEQ, LANES)] = zero16

    pltpu.sync_copy(xtab_hbm, xtab_v)
    pltpu.sync_copy(ytab_hbm, ytab_v)
    pltpu.sync_copy(pos_hbm, pos_v.at[pl.ds(0, SEQ * DIM)])
    pltpu.sync_copy(lane_hbm, lane_v)
    pltpu.sync_copy(
        lidx_hbm.at[pl.ds(base_b, ROWS_PER_W)], lidx_v.at[pl.ds(0, ROWS_PER_W)]
    )

    for r in range(NLANE):
        for k in range(DBLK):
            sl = pl.ds(r * DIM + LANES * k, LANES)
            lane_v[sl] = lane_v[sl] * 10.0

    def tok_copies(pair, tbuf):
        src = pl.ds((base_b + 2 * pair) * SEQ, 2 * SEQ)
        dst = pl.ds(tbuf * TOKBUF, 2 * SEQ)
        return (
            pltpu.make_async_copy(xtok_hbm.at[src], tokx_v.at[dst], sem_t),
            pltpu.make_async_copy(ytok_hbm.at[src], toky_v.at[dst], sem_t),
        )

    def copy_q(q, bb):
        return pltpu.make_async_copy(
            outq[q].at[pl.ds(0, PROWS)],
            out_hbm.at[bb],
            sem_q[q],
        )

    for cp in tok_copies(0, 0):
        cp.start()

    NPAIRS = ROWS_PER_W // 2

    def compute_row(t, row_off, out_ref, b_local):
        l = lidx_v[pl.ds(b_local, LANES)][0]
        lvec = [lane_v[pl.ds(l * DIM + LANES * k, LANES)] for k in range(DBLK)]

        @plsc.parallel_loop(0, NGROUP, unroll=NGROUP)
        def grp(g):
            s0 = g * LANES
            r0 = s0
            txv = tokx_v[pl.ds(t * TOKBUF + row_off + s0, LANES)]
            tyv = toky_v[pl.ds(t * TOKBUF + row_off + s0, LANES)]
            for j in range(0, LANES, 2):
                tx0 = txv[j] * DIM
                ty0 = tyv[j] * DIM
                tx1 = txv[j + 1] * DIM
                ty1 = tyv[j + 1] * DIM
                so0 = (s0 + j) * DIM
                so1 = so0 + DIM
                x0 = [xtab_v[pl.ds(tx0 + LANES * k, LANES)] for k in range(DBLK)]
                y0 = [ytab_v[pl.ds(ty0 + LANES * k, LANES)] for k in range(DBLK)]
                p0 = [pos_v[pl.ds(so0 + LANES * k, LANES)] for k in range(DBLK)]
                x1 = [xtab_v[pl.ds(tx1 + LANES * k, LANES)] for k in range(DBLK)]
                y1 = [ytab_v[pl.ds(ty1 + LANES * k, LANES)] for k in range(DBLK)]
                p1 = [pos_v[pl.ds(so1 + LANES * k, LANES)] for k in range(DBLK)]
                rp = (r0 + j) // 2
                for k in range(DBLK):
                    off = LANES * k
                    out_ref[rp, pl.ds(off, LANES)] = (x0[k] + y0[k]) + (
                        p0[k] + lvec[k]
                    )
                for k in range(DBLK):
                    off = LANES * k
                    out_ref[rp, pl.ds(DIM + off, LANES)] = (x1[k] + y1[k]) + (
                        p1[k] + lvec[k]
                    )

    def do_row(q, b, t, row_off, bb):
        @pl.when(b >= 2)
        def _():
            copy_q(q, bb).wait()

        compute_row(t, row_off, outq[q], b)
        copy_q(q, bb).start()

    def pair_body(p, carry):
        t = lax.bitwise_and(p, 1)
        bb0 = base_b + 2 * p

        for cp in tok_copies(p, t):
            cp.wait()

        @pl.when(p < NPAIRS - 1)
        def _():
            for cp in tok_copies(p + 1, 1 - t):
                cp.start()

        do_row(0, 2 * p, t, 0, bb0)
        do_row(1, 2 * p + 1, t, SEQ, bb0 + 1)
        return carry

    lax.fori_loop(0, NPAIRS, pair_body, 0)

    copy_q(0, base_b + ROWS_PER_W - 2).wait()
    copy_q(1, base_b + ROWS_PER_W - 1).wait()


@jax.jit
def _run(xtok, ytok, lidx, xtab, ytab, pos, lane):
    mesh = plsc.VectorSubcoreMesh(core_axis_name="c", subcore_axis_name="s")
    flat = pl.kernel(
        _body,
        out_type=jax.ShapeDtypeStruct((BATCH, SEQ // 2, DPAD), jnp.float32),
        mesh=mesh,
        scratch_types=[
            pltpu.VMEM((NBINS_X * DIM,), jnp.float32),
            pltpu.VMEM((NY * DIM,), jnp.float32),
            pltpu.VMEM((SEQ_PAD * DIM,), jnp.float32),
            pltpu.VMEM((NLANE * DIM,), jnp.float32),
            pltpu.VMEM((ROWS_PER_W + LANES,), jnp.int32),
            pltpu.VMEM((2 * TOKBUF,), jnp.int32),
            pltpu.VMEM((2 * TOKBUF,), jnp.int32),
            pltpu.VMEM((PBUF, DPAD), jnp.float32),
            pltpu.VMEM((PBUF, DPAD), jnp.float32),
            pltpu.SemaphoreType.DMA,
            pltpu.SemaphoreType.DMA,
            pltpu.SemaphoreType.DMA,
        ],
    )(xtok, ytok, lidx, xtab, ytab, pos, lane)
    return flat.reshape(BATCH, SEQ, DIM)


def kernel(x_tokens, y_tokens, lane_indices, x_table, y_table, pos_table, lane_table):
    return _run(
        x_tokens.astype(jnp.int32).reshape(-1),
        y_tokens.astype(jnp.int32).reshape(-1),
        lane_indices.astype(jnp.int32),
        x_table.reshape(-1),
        y_table.reshape(-1),
        pos_table.reshape(-1),
        lane_table.reshape(-1),
    )


# speedup vs baseline: 1.2392x; 1.2392x over previous
import functools

import jax
import jax.numpy as jnp
from jax import lax
from jax.experimental import pallas as pl
from jax.experimental.pallas import tpu as pltpu
from jax.experimental.pallas import tpu_sc as plsc

BATCH = 4096
SEQ = 200
DIM = 64
NBINS_X = 1000
NY = 201
NLANE = 8

NUM_CORES = 2
NUM_SUBCORES = 16
NUM_WORKERS = NUM_CORES * NUM_SUBCORES
ROWS_PER_W = BATCH // NUM_WORKERS
LANES = 16
DBLK = DIM // LANES
NGROUP = (SEQ + LANES - 1) // LANES
SEQ_PAD = NGROUP * LANES
SPLIT = 96
Q_GLO = (0, 6)
Q_GHI = (6, 13)
Q_SBASE = (0, SPLIT)
Q_ROWS = (SPLIT, SEQ - SPLIT)
Q_BUF = (SPLIT, SEQ_PAD - SPLIT)
TOKBUF = 2 * SEQ + LANES


def _body(
    xtok_hbm,
    ytok_hbm,
    lidx_hbm,
    xtab_hbm,
    ytab_hbm,
    pos_hbm,
    lane_hbm,
    out_hbm,
    xtab_v,
    ytab_v,
    pos_v,
    lane_v,
    lidx_v,
    tokx_v,
    toky_v,
    outq0_v,
    outq1_v,
    sem_q0,
    sem_q1,
    sem_t,
):
    outq = (outq0_v, outq1_v)
    sem_q = (sem_q0, sem_q1)
    wid = lax.axis_index("s") * NUM_CORES + lax.axis_index("c")
    base_b = wid * ROWS_PER_W

    zero16 = jnp.zeros((LANES,), jnp.int32)
    for t in range(2):
        tokx_v[pl.ds(t * TOKBUF + 2 * SEQ, LANES)] = zero16
        toky_v[pl.ds(t * TOKBUF + 2 * SEQ, LANES)] = zero16

    pltpu.sync_copy(xtab_hbm, xtab_v)
    pltpu.sync_copy(ytab_hbm, ytab_v)
    pltpu.sync_copy(pos_hbm, pos_v.at[pl.ds(0, SEQ * DIM)])
    pltpu.sync_copy(lane_hbm, lane_v)
    pltpu.sync_copy(
        lidx_hbm.at[pl.ds(base_b, ROWS_PER_W)], lidx_v.at[pl.ds(0, ROWS_PER_W)]
    )

    for r in range(NLANE):
        for k in range(DBLK):
            sl = pl.ds(r * DIM + LANES * k, LANES)
            lane_v[sl] = lane_v[sl] * 10.0

    def tok_copies(pair, tbuf):
        src = pl.ds((base_b + 2 * pair) * SEQ, 2 * SEQ)
        dst = pl.ds(tbuf * TOKBUF, 2 * SEQ)
        return (
            pltpu.make_async_copy(xtok_hbm.at[src], tokx_v.at[dst], sem_t),
            pltpu.make_async_copy(ytok_hbm.at[src], toky_v.at[dst], sem_t),
        )

    def copy_q(q, bb):
        return pltpu.make_async_copy(
            outq[q].at[pl.ds(0, Q_ROWS[q])],
            out_hbm.at[bb, pl.ds(Q_SBASE[q], Q_ROWS[q])],
            sem_q[q],
        )

    for cp in tok_copies(0, 0):
        cp.start()

    NPAIRS = ROWS_PER_W // 2

    def compute_half(t, row_off, out_ref, b_local, g_lo, g_hi, s_base):
        l = lidx_v[pl.ds(b_local, LANES)][0]
        lvec = [lane_v[pl.ds(l * DIM + LANES * k, LANES)] for k in range(DBLK)]

        @plsc.parallel_loop(g_lo, g_hi, unroll=2)
        def grp(g):
            s0 = g * LANES
            r0 = s0 - s_base
            txv = tokx_v[pl.ds(t * TOKBUF + row_off + s0, LANES)]
            tyv = toky_v[pl.ds(t * TOKBUF + row_off + s0, LANES)]
            for j in range(0, LANES, 2):
                tx0 = txv[j] * DIM
                ty0 = tyv[j] * DIM
                tx1 = txv[j + 1] * DIM
                ty1 = tyv[j + 1] * DIM
                so0 = (s0 + j) * DIM
                so1 = so0 + DIM
                x0 = [xtab_v[pl.ds(tx0 + LANES * k, LANES)] for k in range(DBLK)]
                y0 = [ytab_v[pl.ds(ty0 + LANES * k, LANES)] for k in range(DBLK)]
                p0 = [pos_v[pl.ds(so0 + LANES * k, LANES)] for k in range(DBLK)]
                x1 = [xtab_v[pl.ds(tx1 + LANES * k, LANES)] for k in range(DBLK)]
                y1 = [ytab_v[pl.ds(ty1 + LANES * k, LANES)] for k in range(DBLK)]
                p1 = [pos_v[pl.ds(so1 + LANES * k, LANES)] for k in range(DBLK)]
                for k in range(DBLK):
                    off = LANES * k
                    out_ref[r0 + j, pl.ds(off, LANES)] = (x0[k] + y0[k]) + (
                        p0[k] + lvec[k]
                    )
                for k in range(DBLK):
                    off = LANES * k
                    out_ref[r0 + j + 1, pl.ds(off, LANES)] = (x1[k] + y1[k]) + (
                        p1[k] + lvec[k]
                    )

    def do_half(q, b, t, row_off, bb):
        @pl.when(b >= 1)
        def _():
            copy_q(q, bb).wait()

        compute_half(t, row_off, outq[q], b, Q_GLO[q], Q_GHI[q], Q_SBASE[q])
        copy_q(q, bb).start()

    def pair_body(p, carry):
        t = lax.bitwise_and(p, 1)
        bb0 = base_b + 2 * p

        for cp in tok_copies(p, t):
            cp.wait()

        @pl.when(p < NPAIRS - 1)
        def _():
            for cp in tok_copies(p + 1, 1 - t):
                cp.start()

        do_half(0, 2 * p, t, 0, bb0)
        do_half(1, 2 * p, t, 0, bb0)
        do_half(0, 2 * p + 1, t, SEQ, bb0 + 1)
        do_half(1, 2 * p + 1, t, SEQ, bb0 + 1)
        return carry

    lax.fori_loop(0, NPAIRS, pair_body, 0)

    copy_q(0, base_b + ROWS_PER_W - 1).wait()
    copy_q(1, base_b + ROWS_PER_W - 1).wait()


@jax.jit
def _run(xtok, ytok, lidx, xtab, ytab, pos, lane):
    mesh = plsc.VectorSubcoreMesh(core_axis_name="c", subcore_axis_name="s")
    flat = pl.kernel(
        _body,
        out_type=jax.ShapeDtypeStruct((BATCH, SEQ, DIM), jnp.float32),
        mesh=mesh,
        scratch_types=[
            pltpu.VMEM((NBINS_X * DIM,), jnp.float32),
            pltpu.VMEM((NY * DIM,), jnp.float32),
            pltpu.VMEM((SEQ_PAD * DIM,), jnp.float32),
            pltpu.VMEM((NLANE * DIM,), jnp.float32),
            pltpu.VMEM((ROWS_PER_W + LANES,), jnp.int32),
            pltpu.VMEM((2 * TOKBUF,), jnp.int32),
            pltpu.VMEM((2 * TOKBUF,), jnp.int32),
            pltpu.VMEM((Q_BUF[0], DIM), jnp.float32),
            pltpu.VMEM((Q_BUF[1], DIM), jnp.float32),
            pltpu.SemaphoreType.DMA,
            pltpu.SemaphoreType.DMA,
            pltpu.SemaphoreType.DMA,
        ],
    )(xtok, ytok, lidx, xtab, ytab, pos, lane)
    return flat


def kernel(x_tokens, y_tokens, lane_indices, x_table, y_table, pos_table, lane_table):
    return _run(
        x_tokens.astype(jnp.int32).reshape(-1),
        y_tokens.astype(jnp.int32).reshape(-1),
        lane_indices.astype(jnp.int32),
        x_table.reshape(-1),
        y_table.reshape(-1),
        pos_table.reshape(-1),
        lane_table.reshape(-1),
    )


# speedup vs baseline: 1.8059x; 1.4573x over previous
import functools

import jax
import jax.numpy as jnp
from jax import lax
from jax.experimental import pallas as pl
from jax.experimental.pallas import tpu as pltpu
from jax.experimental.pallas import tpu_sc as plsc

BATCH = 4096
SEQ = 200
DIM = 64
NBINS_X = 1000
NY = 201
NLANE = 8

NUM_CORES = 2
NUM_SUBCORES = 16
NUM_WORKERS = NUM_CORES * NUM_SUBCORES
ROWS_PER_W = BATCH // NUM_WORKERS
LANES = 16
DBLK = DIM // LANES
NGROUP = (SEQ + LANES - 1) // LANES
SEQ_PAD = NGROUP * LANES
DPAD = 128
PROWS = SEQ // 2
PBUF = SEQ_PAD // 2
TOKBUF = 2 * SEQ + LANES


def _body(
    xtok_hbm,
    ytok_hbm,
    lidx_hbm,
    xtab_hbm,
    ytab_hbm,
    pos_hbm,
    lane_hbm,
    out_hbm,
    xtab_v,
    ytab_v,
    pos_v,
    lane_v,
    lidx_v,
    tokx_v,
    toky_v,
    outq0_v,
    outq1_v,
    sem_q0,
    sem_q1,
    sem_t,
):
    outq = (outq0_v, outq1_v)
    sem_q = (sem_q0, sem_q1)
    wid = lax.axis_index("s") * NUM_CORES + lax.axis_index("c")
    base_b = wid * ROWS_PER_W

    zero16 = jnp.zeros((LANES,), jnp.int32)
    for t in range(2):
        tokx_v[pl.ds(t * TOKBUF + 2 * SEQ, LANES)] = zero16
        toky_v[pl.ds(t * TOKBUF + 2 * SEQ, LANES)] = zero16

    pltpu.sync_copy(xtab_hbm, xtab_v)
    pltpu.sync_copy(ytab_hbm, ytab_v)
    pltpu.sync_copy(pos_hbm, pos_v.at[pl.ds(0, SEQ * DIM)])
    pltpu.sync_copy(lane_hbm, lane_v)
    pltpu.sync_copy(
        lidx_hbm.at[pl.ds(base_b, ROWS_PER_W)], lidx_v.at[pl.ds(0, ROWS_PER_W)]
    )

    for r in range(NLANE):
        for k in range(DBLK):
            sl = pl.ds(r * DIM + LANES * k, LANES)
            lane_v[sl] = lane_v[sl] * 10.0

    def tok_copies(pair, tbuf):
        src = pl.ds((base_b + 2 * pair) * SEQ, 2 * SEQ)
        dst = pl.ds(tbuf * TOKBUF, 2 * SEQ)
        return (
            pltpu.make_async_copy(xtok_hbm.at[src], tokx_v.at[dst], sem_t),
            pltpu.make_async_copy(ytok_hbm.at[src], toky_v.at[dst], sem_t),
        )

    def copy_q(q, bb):
        return pltpu.make_async_copy(
            outq[q].at[pl.ds(0, PROWS)],
            out_hbm.at[bb],
            sem_q[q],
        )

    for cp in tok_copies(0, 0):
        cp.start()

    NPAIRS = ROWS_PER_W // 2

    def compute_row(t, row_off, out_ref, b_local):
        l = lidx_v[pl.ds(b_local, LANES)][0]
        lvec = [lane_v[pl.ds(l * DIM + LANES * k, LANES)] for k in range(DBLK)]

        @plsc.parallel_loop(0, NGROUP, unroll=2)
        def grp(g):
            s0 = g * LANES
            r0 = s0
            txv = tokx_v[pl.ds(t * TOKBUF + row_off + s0, LANES)]
            tyv = toky_v[pl.ds(t * TOKBUF + row_off + s0, LANES)]
            for j in range(0, LANES, 2):
                tx0 = txv[j] * DIM
                ty0 = tyv[j] * DIM
                tx1 = txv[j + 1] * DIM
                ty1 = tyv[j + 1] * DIM
                so0 = (s0 + j) * DIM
                so1 = so0 + DIM
                x0 = [xtab_v[pl.ds(tx0 + LANES * k, LANES)] for k in range(DBLK)]
                y0 = [ytab_v[pl.ds(ty0 + LANES * k, LANES)] for k in range(DBLK)]
                p0 = [pos_v[pl.ds(so0 + LANES * k, LANES)] for k in range(DBLK)]
                x1 = [xtab_v[pl.ds(tx1 + LANES * k, LANES)] for k in range(DBLK)]
                y1 = [ytab_v[pl.ds(ty1 + LANES * k, LANES)] for k in range(DBLK)]
                p1 = [pos_v[pl.ds(so1 + LANES * k, LANES)] for k in range(DBLK)]
                rp = (r0 + j) // 2
                for k in range(DBLK):
                    off = LANES * k
                    out_ref[rp, pl.ds(off, LANES)] = (x0[k] + y0[k]) + (
                        p0[k] + lvec[k]
                    )
                for k in range(DBLK):
                    off = LANES * k
                    out_ref[rp, pl.ds(DIM + off, LANES)] = (x1[k] + y1[k]) + (
                        p1[k] + lvec[k]
                    )

    def do_row(q, b, t, row_off, bb):
        @pl.when(b >= 2)
        def _():
            copy_q(q, bb).wait()

        compute_row(t, row_off, outq[q], b)
        copy_q(q, bb).start()

    def pair_body(p, carry):
        t = lax.bitwise_and(p, 1)
        bb0 = base_b + 2 * p

        for cp in tok_copies(p, t):
            cp.wait()

        @pl.when(p < NPAIRS - 1)
        def _():
            for cp in tok_copies(p + 1, 1 - t):
                cp.start()

        do_row(0, 2 * p, t, 0, bb0)
        do_row(1, 2 * p + 1, t, SEQ, bb0 + 1)
        return carry

    lax.fori_loop(0, NPAIRS, pair_body, 0)

    copy_q(0, base_b + ROWS_PER_W - 2).wait()
    copy_q(1, base_b + ROWS_PER_W - 1).wait()


@jax.jit
def _run(xtok, ytok, lidx, xtab, ytab, pos, lane):
    mesh = plsc.VectorSubcoreMesh(core_axis_name="c", subcore_axis_name="s")
    flat = pl.kernel(
        _body,
        out_type=jax.ShapeDtypeStruct((BATCH, SEQ // 2, DPAD), jnp.float32),
        mesh=mesh,
        scratch_types=[
            pltpu.VMEM((NBINS_X * DIM,), jnp.float32),
            pltpu.VMEM((NY * DIM,), jnp.float32),
            pltpu.VMEM((SEQ_PAD * DIM,), jnp.float32),
            pltpu.VMEM((NLANE * DIM,), jnp.float32),
            pltpu.VMEM((ROWS_PER_W + LANES,), jnp.int32),
            pltpu.VMEM((2 * TOKBUF,), jnp.int32),
            pltpu.VMEM((2 * TOKBUF,), jnp.int32),
            pltpu.VMEM((PBUF, DPAD), jnp.float32),
            pltpu.VMEM((PBUF, DPAD), jnp.float32),
            pltpu.SemaphoreType.DMA,
            pltpu.SemaphoreType.DMA,
            pltpu.SemaphoreType.DMA,
        ],
    )(xtok, ytok, lidx, xtab, ytab, pos, lane)
    return flat.reshape(BATCH, SEQ, DIM)


def kernel(x_tokens, y_tokens, lane_indices, x_table, y_table, pos_table, lane_table):
    return _run(
        x_tokens.astype(jnp.int32).reshape(-1),
        y_tokens.astype(jnp.int32).reshape(-1),
        lane_indices.astype(jnp.int32),
        x_table.reshape(-1),
        y_table.reshape(-1),
        pos_table.reshape(-1),
        lane_table.reshape(-1),
    )
